# baseline (speedup 1.0000x reference)
import functools

import jax
import jax.numpy as jnp
from jax.experimental import pallas as pl


_TAU = 0.9
_P = 0.8
_MIN_K = 1
_E = 16


def _router_block(patch_ref, convw_ref, convb_ref, fcw_ref, fcb_ref, out_ref):
    x = patch_ref[...]
    w = convw_ref[...]
    y = jax.lax.dot_general(
        w, x, (((1,), (1,)), ((), ())),
        preferred_element_type=jnp.float32)
    y = jnp.maximum(y + convb_ref[...][:, :, None], 0.0)
    pooled = jnp.mean(y, axis=2)
    logits = jax.lax.dot_general(
        fcw_ref[...], pooled, (((1,), (0,)), ((), ())),
        preferred_element_type=jnp.float32) + fcb_ref[...]

    z = logits * (1.0 / _TAU)
    z = z - jnp.max(z, axis=0, keepdims=True)
    e = jnp.exp(z)
    probs = e / jnp.sum(e, axis=0, keepdims=True)

    cums = jnp.zeros_like(probs)
    rank = jnp.zeros_like(probs)
    i_idx = jax.lax.broadcasted_iota(jnp.int32, (_E, 1), 0)
    for j in range(_E):
        vj = probs[j:j + 1, :]
        prec_incl = (vj > probs) | ((vj == probs) & (j <= i_idx))
        cums = cums + jnp.where(prec_incl, vj, 0.0)
        rank = rank + jnp.where(prec_incl, 1.0, 0.0)
    keep = (cums <= _P) | (rank - 1.0 < _MIN_K)
    masked = jnp.where(keep, probs, 0.0)
    denom = jnp.clip(jnp.sum(masked, axis=0, keepdims=True), 1e-10, None)
    out_ref[...] = masked / denom


@functools.partial(jax.jit, static_argnames=())
def _run(patch, conv_w, conv_b, fc_w, fc_b):
    B = patch.shape[0]
    BC = 128
    x = patch.reshape(B, 196, 64)
    conv_b2 = conv_b.reshape(128, 1)
    fc_b2 = fc_b.reshape(_E, 1)
    return pl.pallas_call(
        _router_block,
        grid=(B // BC,),
        in_specs=[
            pl.BlockSpec((BC, 196, 64), lambda i: (i, 0, 0)),
            pl.BlockSpec((128, 196), lambda i: (0, 0)),
            pl.BlockSpec((128, 1), lambda i: (0, 0)),
            pl.BlockSpec((_E, 128), lambda i: (0, 0)),
            pl.BlockSpec((_E, 1), lambda i: (0, 0)),
        ],
        out_specs=pl.BlockSpec((_E, BC), lambda i: (0, i)),
        out_shape=jax.ShapeDtypeStruct((_E, B), jnp.float32),
    )(x, conv_w, conv_b2, fc_w, fc_b2)


def kernel(patch, conv_w, conv_b, fc_w, fc_b, layer_idx, threshold):
    return _run(patch, conv_w, conv_b, fc_w, fc_b).T


# speedup vs baseline: 1.9383x; 1.9383x over previous
import functools

import jax
import jax.numpy as jnp
from jax.experimental import pallas as pl


_TAU = 0.9
_P = 0.8
_MIN_K = 1
_E = 16


def _router_block(x2_ref, we_ref, wo_ref, convb_ref, fcw_ref, fcb_ref, out_ref):
    x2 = x2_ref[...]
    x2t = jnp.transpose(x2, (0, 2, 1))
    xe = x2t[:, :64, :]
    xo = x2t[:, 64:, :]
    y = jax.lax.dot_general(
        xe, we_ref[...], (((2,), (0,)), ((), ())),
        preferred_element_type=jnp.float32)
    y = y + jax.lax.dot_general(
        xo, wo_ref[...], (((2,), (0,)), ((), ())),
        preferred_element_type=jnp.float32)
    z = jnp.maximum(y + convb_ref[...][None, :, :], 0.0)
    pooled = jnp.mean(z, axis=1)
    logits_t = jax.lax.dot_general(
        fcw_ref[...], pooled, (((1,), (1,)), ((), ())),
        preferred_element_type=jnp.float32) + fcb_ref[...]

    zl = logits_t * (1.0 / _TAU)
    zl = zl - jnp.max(zl, axis=0, keepdims=True)
    e = jnp.exp(zl)
    probs = e / jnp.sum(e, axis=0, keepdims=True)

    cums = jnp.zeros_like(probs)
    rank = jnp.zeros_like(probs)
    i_idx = jax.lax.broadcasted_iota(jnp.int32, (_E, 1), 0)
    for j in range(_E):
        vj = probs[j:j + 1, :]
        prec_incl = (vj > probs) | ((vj == probs) & (j <= i_idx))
        cums = cums + jnp.where(prec_incl, vj, 0.0)
        rank = rank + jnp.where(prec_incl, 1.0, 0.0)
    keep = (cums <= _P) | (rank - 1.0 < _MIN_K)
    masked = jnp.where(keep, probs, 0.0)
    denom = jnp.clip(jnp.sum(masked, axis=0, keepdims=True), 1e-10, None)
    out_ref[...] = masked / denom


@functools.partial(jax.jit, static_argnames=())
def _run(patch, conv_w, conv_b, fc_w, fc_b):
    B = patch.shape[0]
    BC = 128
    x2 = patch.reshape(B, 98, 128)
    we = conv_w[:, 0::2].T
    wo = conv_w[:, 1::2].T
    conv_b2 = conv_b.reshape(1, 128)
    fc_b2 = fc_b.reshape(_E, 1)
    out_t = pl.pallas_call(
        _router_block,
        grid=(B // BC,),
        in_specs=[
            pl.BlockSpec((BC, 98, 128), lambda i: (i, 0, 0)),
            pl.BlockSpec((98, 128), lambda i: (0, 0)),
            pl.BlockSpec((98, 128), lambda i: (0, 0)),
            pl.BlockSpec((1, 128), lambda i: (0, 0)),
            pl.BlockSpec((_E, 128), lambda i: (0, 0)),
            pl.BlockSpec((_E, 1), lambda i: (0, 0)),
        ],
        out_specs=pl.BlockSpec((_E, BC), lambda i: (0, i)),
        out_shape=jax.ShapeDtypeStruct((_E, B), jnp.float32),
    )(x2, we, wo, conv_b2, fc_w, fc_b2)
    return out_t.T


def kernel(patch, conv_w, conv_b, fc_w, fc_b, layer_idx, threshold):
    return _run(patch, conv_w, conv_b, fc_w, fc_b)


# speedup vs baseline: 4.8487x; 2.5015x over previous
import functools

import jax
import jax.numpy as jnp
from jax.experimental import pallas as pl


_TAU = 0.9
_P = 0.8
_MIN_K = 1
_E = 16


def _router_block(x_ref, web_ref, wob_ref, convb_ref, fcw_ref, fcb_ref, out_ref):
    x = x_ref[...]
    xt = jnp.transpose(x, (1, 0, 2))
    xe = xt[:64]
    xo = xt[64:]
    y = jax.lax.dot_general(
        web_ref[...], xe, (((2,), (1,)), ((0,), (0,))),
        preferred_element_type=jnp.float32)
    y = y + jax.lax.dot_general(
        wob_ref[...], xo, (((2,), (1,)), ((0,), (0,))),
        preferred_element_type=jnp.float32)
    z = jnp.maximum(y + convb_ref[...][None, :, :], 0.0)
    pooled = jnp.mean(z, axis=0)
    logits = jax.lax.dot_general(
        fcw_ref[...], pooled, (((1,), (0,)), ((), ())),
        preferred_element_type=jnp.float32) + fcb_ref[...]

    zl = logits * (1.0 / _TAU)
    zl = zl - jnp.max(zl, axis=0, keepdims=True)
    e = jnp.exp(zl)
    probs = e / jnp.sum(e, axis=0, keepdims=True)

    cums = jnp.zeros_like(probs)
    rank = jnp.zeros_like(probs)
    i_idx = jax.lax.broadcasted_iota(jnp.int32, (_E, 1), 0)
    for j in range(_E):
        vj = probs[j:j + 1, :]
        prec_incl = (vj > probs) | ((vj == probs) & (j <= i_idx))
        cums = cums + jnp.where(prec_incl, vj, 0.0)
        rank = rank + jnp.where(prec_incl, 1.0, 0.0)
    keep = (cums <= _P) | (rank - 1.0 < _MIN_K)
    masked = jnp.where(keep, probs, 0.0)
    denom = jnp.clip(jnp.sum(masked, axis=0, keepdims=True), 1e-10, None)
    out_ref[...] = masked / denom


@functools.partial(jax.jit, static_argnames=())
def _run(patch, conv_w, conv_b, fc_w, fc_b):
    B = patch.shape[0]
    BC = 128
    x3 = jnp.transpose(patch.reshape(B, 98, 128), (1, 2, 0))
    we = conv_w[:, 0::2]
    wo = conv_w[:, 1::2]
    web = jnp.broadcast_to(we[None], (64, 128, 98))
    wob = jnp.broadcast_to(wo[None], (64, 128, 98))
    conv_b2 = conv_b.reshape(128, 1)
    fc_b2 = fc_b.reshape(_E, 1)
    out_t = pl.pallas_call(
        _router_block,
        grid=(B // BC,),
        in_specs=[
            pl.BlockSpec((98, 128, BC), lambda i: (0, 0, i)),
            pl.BlockSpec((64, 128, 98), lambda i: (0, 0, 0)),
            pl.BlockSpec((64, 128, 98), lambda i: (0, 0, 0)),
            pl.BlockSpec((128, 1), lambda i: (0, 0)),
            pl.BlockSpec((_E, 128), lambda i: (0, 0)),
            pl.BlockSpec((_E, 1), lambda i: (0, 0)),
        ],
        out_specs=pl.BlockSpec((_E, BC), lambda i: (0, i)),
        out_shape=jax.ShapeDtypeStruct((_E, B), jnp.float32),
    )(x3, web, wob, conv_b2, fc_w, fc_b2)
    return out_t.T


def kernel(patch, conv_w, conv_b, fc_w, fc_b, layer_idx, threshold):
    return _run(patch, conv_w, conv_b, fc_w, fc_b)


# speedup vs baseline: 5.2978x; 1.0926x over previous
import functools

import jax
import jax.numpy as jnp
from jax.experimental import pallas as pl


_TAU = 0.9
_P = 0.8
_MIN_K = 1
_E = 16


def _router_block(x_ref, web_ref, wob_ref, convb_ref, fcw_ref, fcb_ref, out_ref):
    x = x_ref[...]
    xt = jnp.transpose(x, (1, 0, 2))
    xe = xt[:64]
    xo = xt[64:]
    y = jax.lax.dot_general(
        web_ref[...], xe, (((2,), (1,)), ((0,), (0,))),
        preferred_element_type=jnp.float32)
    y = y + jax.lax.dot_general(
        wob_ref[...], xo, (((2,), (1,)), ((0,), (0,))),
        preferred_element_type=jnp.float32)
    z = jnp.maximum(y + convb_ref[...][None, :, :], 0.0)
    pooled = jnp.mean(z, axis=0)
    logits = jax.lax.dot_general(
        fcw_ref[...], pooled, (((1,), (0,)), ((), ())),
        preferred_element_type=jnp.float32) + fcb_ref[...]

    zl = logits * (1.0 / _TAU)
    zl = zl - jnp.max(zl, axis=0, keepdims=True)
    e = jnp.exp(zl)
    probs = e / jnp.sum(e, axis=0, keepdims=True)

    cums = jnp.zeros_like(probs)
    rank = jnp.zeros_like(probs)
    i_idx = jax.lax.broadcasted_iota(jnp.int32, (_E, 1), 0)
    for j in range(_E):
        vj = probs[j:j + 1, :]
        prec_incl = (vj > probs) | ((vj == probs) & (j <= i_idx))
        cums = cums + jnp.where(prec_incl, vj, 0.0)
        rank = rank + jnp.where(prec_incl, 1.0, 0.0)
    keep = (cums <= _P) | (rank - 1.0 < _MIN_K)
    masked = jnp.where(keep, probs, 0.0)
    denom = jnp.clip(jnp.sum(masked, axis=0, keepdims=True), 1e-10, None)
    out_ref[...] = masked / denom


@functools.partial(jax.jit, static_argnames=())
def _run(patch, conv_w, conv_b, fc_w, fc_b):
    B = patch.shape[0]
    BC = 256
    x3 = jnp.transpose(patch.reshape(B, 98, 128), (1, 2, 0))
    we = conv_w[:, 0::2]
    wo = conv_w[:, 1::2]
    web = jnp.broadcast_to(we[None], (64, 128, 98))
    wob = jnp.broadcast_to(wo[None], (64, 128, 98))
    conv_b2 = conv_b.reshape(128, 1)
    fc_b2 = fc_b.reshape(_E, 1)
    out_t = pl.pallas_call(
        _router_block,
        grid=(B // BC,),
        in_specs=[
            pl.BlockSpec((98, 128, BC), lambda i: (0, 0, i)),
            pl.BlockSpec((64, 128, 98), lambda i: (0, 0, 0)),
            pl.BlockSpec((64, 128, 98), lambda i: (0, 0, 0)),
            pl.BlockSpec((128, 1), lambda i: (0, 0)),
            pl.BlockSpec((_E, 128), lambda i: (0, 0)),
            pl.BlockSpec((_E, 1), lambda i: (0, 0)),
        ],
        out_specs=pl.BlockSpec((_E, BC), lambda i: (0, i)),
        out_shape=jax.ShapeDtypeStruct((_E, B), jnp.float32),
    )(x3, web, wob, conv_b2, fc_w, fc_b2)
    return out_t.T


def kernel(patch, conv_w, conv_b, fc_w, fc_b, layer_idx, threshold):
    return _run(patch, conv_w, conv_b, fc_w, fc_b)
